# baseline (speedup 1.0000x reference)
import jax
import jax.numpy as jnp
from jax import lax
from jax.experimental import pallas as pl
from jax.experimental.pallas import tpu as pltpu
from jax.experimental.pallas import tpu_sc as plsc

NSEL = 300
NCLS = 91
QC = 81900
QC_PAD = 81920
NVEC = QC_PAD // 16
SHIFT = 16
NBUCKET = 16384
NBVEC = NBUCKET // 16
CAND = 512
CVEC = CAND // 16
OUT_PAD = 320
NROW = 64


def _sig_body(x_ref, o_ref):
    o_ref[:, :QC] = jax.nn.sigmoid(x_ref[...])
    o_ref[:, QC:] = jnp.zeros((8, QC_PAD - QC), jnp.float32)


def _sc_topk(prob_hbm, boxes_hbm, scale_hbm, scores_hbm, labels_hbm,
             boxeso_hbm, row_v, hist_v, ck_v, ci_v, boxes_v, scale_v,
             sco_v, lab_v, bxo_v):
    wid = lax.axis_index("s") * 2 + lax.axis_index("c")
    iota = lax.iota(jnp.int32, 16)
    zeros16 = iota * 0
    ones16 = zeros16 + 1

    def do_row(rr, _):
        row = wid * 2 + rr
        pltpu.sync_copy(prob_hbm.at[pl.ds(row * 640, 640)], row_v)
        pltpu.sync_copy(boxes_hbm.at[row], boxes_v)
        pltpu.sync_copy(scale_hbm.at[row], scale_v)

        @plsc.parallel_loop(0, NBVEC, unroll=8)
        def _zero(j):
            hist_v[pl.ds(j * 16, 16)] = zeros16

        @plsc.parallel_loop(0, NVEC, unroll=8)
        def _hist(i):
            k = plsc.bitcast(row_v[i >> 3, pl.ds((i & 7) * 16, 16)], jnp.int32)
            plsc.addupdate_scatter(hist_v, [k >> SHIFT], ones16)

        def thr_cond(carry):
            jr, acc, _ = carry
            return (acc < NSEL) & (jr >= 0)

        def thr_body(carry):
            jr, acc, bstar = carry
            h = hist_v[pl.ds(jr * 16, 16)]
            s = jnp.sum(h)
            rc = plsc.cumsum(lax.rev(h, (0,)))
            f = jnp.max(plsc.all_reduce_ffs((acc + rc) >= NSEL))
            cand_b = jr * 16 + 15 - f
            cross = (acc + s) >= NSEL
            return (jr - 1, acc + s, jnp.where(cross, cand_b, bstar))

        _, _, bstar = lax.while_loop(
            thr_cond, thr_body,
            (jnp.int32(NBVEC - 1), jnp.int32(0), jnp.int32(0)))

        @plsc.parallel_loop(0, CVEC, unroll=4)
        def _init(j):
            ck_v[pl.ds(j * 16, 16)] = zeros16 - 1
            ci_v[pl.ds(j * 16, 16)] = zeros16

        @plsc.parallel_loop(0, NVEC, unroll=4, carry=jnp.int32(0))
        def _compact(i, off):
            k = plsc.bitcast(row_v[i >> 3, pl.ds((i & 7) * 16, 16)], jnp.int32)
            m = (k >> SHIFT) >= bstar
            offc = jnp.minimum(off, CAND - 16)
            plsc.store_compressed(ck_v.at[pl.ds(offc, 16)], k, mask=m)
            plsc.store_compressed(ci_v.at[pl.ds(offc, 16)], iota + i * 16,
                                  mask=m)
            return off + jnp.max(plsc.all_reduce_population_count(m))

        for st in range(1, 10):
            kk = 1 << st
            j = kk >> 1
            while j >= 1:
                if j >= 16:
                    jv = j // 16

                    def _cross(t, c, jv=jv, kk=kk):
                        q = t // jv
                        v = q * (2 * jv) + (t - q * jv)
                        p = v + jv
                        ka = ck_v[pl.ds(v * 16, 16)]
                        ia = ci_v[pl.ds(v * 16, 16)]
                        kb = ck_v[pl.ds(p * 16, 16)]
                        ib = ci_v[pl.ds(p * 16, 16)]
                        a_first = (ka > kb) | ((ka == kb) & (ia < ib))
                        dirf = ((zeros16 + v * 16) & kk) == 0
                        keep = jnp.where(dirf, a_first, ~a_first)
                        ck_v[pl.ds(v * 16, 16)] = jnp.where(keep, ka, kb)
                        ci_v[pl.ds(v * 16, 16)] = jnp.where(keep, ia, ib)
                        ck_v[pl.ds(p * 16, 16)] = jnp.where(keep, kb, ka)
                        ci_v[pl.ds(p * 16, 16)] = jnp.where(keep, ib, ia)
                        return c
                    lax.fori_loop(0, CVEC // 2, _cross, 0)
                else:
                    perm = iota ^ j
                    lower = (iota & j) == 0

                    def _intra(v, c, j=j, kk=kk, perm=perm, lower=lower):
                        ks = ck_v[pl.ds(v * 16, 16)]
                        is_ = ci_v[pl.ds(v * 16, 16)]
                        ko = plsc.load_gather(ck_v, [v * 16 + perm])
                        io = plsc.load_gather(ci_v, [v * 16 + perm])
                        s_first = (ks > ko) | ((ks == ko) & (is_ < io))
                        dirf = ((iota + v * 16) & kk) == 0
                        keep = jnp.where(lower == dirf, s_first, ~s_first)
                        ck_v[pl.ds(v * 16, 16)] = jnp.where(keep, ks, ko)
                        ci_v[pl.ds(v * 16, 16)] = jnp.where(keep, is_, io)
                        return c
                    lax.fori_loop(0, CVEC, _intra, 0)
                j >>= 1

        s0 = scale_v[0]
        s1 = scale_v[1]
        s2 = scale_v[2]
        s3 = scale_v[3]

        def _emit(jj, c):
            kj = ck_v[pl.ds(jj * 16, 16)]
            ij = ci_v[pl.ds(jj * 16, 16)]
            sco_v[pl.ds(jj * 16, 16)] = plsc.bitcast(kj, jnp.float32)
            bq = ij // NCLS
            lab_v[pl.ds(jj * 16, 16)] = ij - bq * NCLS
            bqc = jnp.minimum(bq, QC // NCLS - 1)
            b4 = bqc * 4
            cx = plsc.load_gather(boxes_v, [b4])
            cy = plsc.load_gather(boxes_v, [b4 + 1])
            w = plsc.load_gather(boxes_v, [b4 + 2])
            h = plsc.load_gather(boxes_v, [b4 + 3])
            bxo_v[0, pl.ds(jj * 16, 16)] = (cx - 0.5 * w) * s0
            bxo_v[1, pl.ds(jj * 16, 16)] = (cy - 0.5 * h) * s1
            bxo_v[2, pl.ds(jj * 16, 16)] = (cx + 0.5 * w) * s2
            bxo_v[3, pl.ds(jj * 16, 16)] = (cy + 0.5 * h) * s3
            return c
        lax.fori_loop(0, OUT_PAD // 16, _emit, 0)

        pltpu.sync_copy(sco_v, scores_hbm.at[row])
        pltpu.sync_copy(lab_v, labels_hbm.at[row])
        pltpu.sync_copy(bxo_v, boxeso_hbm.at[row])
        return 0

    lax.fori_loop(0, 2, do_row, 0)


def kernel(obj_logits, obj_boxes, target_sizes):
    B, Q, C = obj_logits.shape
    flat = obj_logits.reshape(B, Q * C)
    prob = pl.pallas_call(
        _sig_body,
        out_shape=jax.ShapeDtypeStruct((B, QC_PAD), jnp.float32),
        grid=(B // 8,),
        in_specs=[pl.BlockSpec((8, QC), lambda i: (i, 0))],
        out_specs=pl.BlockSpec((8, QC_PAD), lambda i: (i, 0)),
    )(flat)

    img_h = target_sizes[:, 0].astype(jnp.float32)
    img_w = target_sizes[:, 1].astype(jnp.float32)
    scale = jnp.stack([img_w, img_h, img_w, img_h], axis=1)
    scale16 = jnp.broadcast_to(scale[:, :, None], (B, 4, 16))

    sc = pl.kernel(
        _sc_topk,
        out_type=[
            jax.ShapeDtypeStruct((NROW, OUT_PAD), jnp.float32),
            jax.ShapeDtypeStruct((NROW, OUT_PAD), jnp.int32),
            jax.ShapeDtypeStruct((NROW, 4, OUT_PAD), jnp.float32),
        ],
        mesh=plsc.VectorSubcoreMesh(core_axis_name="c", subcore_axis_name="s"),
        compiler_params=pltpu.CompilerParams(needs_layout_passes=False),
        scratch_types=[
            pltpu.VMEM((QC_PAD // 128, 128), jnp.float32),
            pltpu.VMEM((NBUCKET,), jnp.int32),
            pltpu.VMEM((CAND,), jnp.int32),
            pltpu.VMEM((CAND,), jnp.int32),
            pltpu.VMEM((4 * Q,), jnp.float32),
            pltpu.VMEM((4, 16), jnp.float32),
            pltpu.VMEM((OUT_PAD,), jnp.float32),
            pltpu.VMEM((OUT_PAD,), jnp.int32),
            pltpu.VMEM((4, OUT_PAD), jnp.float32),
        ],
    )
    scores_p, labels_p, boxes_p = sc(prob.reshape(B * QC_PAD // 128, 128), obj_boxes.reshape(B, 4 * Q), scale16)
    return (scores_p[:, :NSEL], labels_p[:, :NSEL],
            boxes_p.transpose(0, 2, 1)[:, :NSEL, :])


# speedup vs baseline: 1.1218x; 1.1218x over previous
import jax
import jax.numpy as jnp
from jax import lax
from jax.experimental import pallas as pl
from jax.experimental.pallas import tpu as pltpu
from jax.experimental.pallas import tpu_sc as plsc

NSEL = 300
NCLS = 91
QC = 81900
QC_PAD = 81920
NVEC = QC_PAD // 16
SHIFT = 16
NBUCKET = 16384
NBVEC = NBUCKET // 16
CAND = 512
CVEC = CAND // 16
OUT_PAD = 384
NROW = 64
BOX_PAD = 3712


def _sig_body(x_ref, o_ref):
    s = jax.nn.sigmoid(x_ref[...])
    o_ref[:, :639, :] = s[:, :639 * 128].reshape(8, 639, 128)
    o_ref[:, 639, :108] = s[:, 639 * 128:]
    o_ref[:, 639, 108:] = jnp.zeros((8, 20), jnp.float32)


def _sc_topk(prob_hbm, boxes_hbm, scale_hbm, scores_hbm, labels_hbm,
             boxeso_hbm, row_v, hist_v, ck_v, ci_v, boxes_v, scale_v,
             sco_v, lab_v, bxo_v):
    wid = lax.axis_index("s") * 2 + lax.axis_index("c")
    iota = lax.iota(jnp.int32, 16)
    zeros16 = iota * 0
    ones16 = zeros16 + 1

    def do_row(rr, _):
        row = wid * 2 + rr
        pltpu.sync_copy(prob_hbm.at[row], row_v)
        pltpu.sync_copy(boxes_hbm.at[row], boxes_v)
        pltpu.sync_copy(scale_hbm.at[row], scale_v)

        @plsc.parallel_loop(0, NBVEC, unroll=8)
        def _zero(j):
            hist_v[pl.ds(j * 16, 16)] = zeros16

        @plsc.parallel_loop(0, NVEC, unroll=8)
        def _hist(i):
            k = plsc.bitcast(row_v[i >> 3, pl.ds((i & 7) * 16, 16)],
                             jnp.int32)
            plsc.addupdate_scatter(hist_v, [k >> SHIFT], ones16)

        def thr_cond(carry):
            jr, acc, _ = carry
            return (acc < NSEL) & (jr >= 0)

        def thr_body(carry):
            jr, acc, bstar = carry
            h = hist_v[pl.ds(jr * 16, 16)]
            s = jnp.sum(h)
            rc = plsc.cumsum(lax.rev(h, (0,)))
            f = jnp.max(plsc.all_reduce_ffs((acc + rc) >= NSEL))
            cand_b = jr * 16 + 15 - f
            cross = (acc + s) >= NSEL
            return (jr - 1, acc + s, jnp.where(cross, cand_b, bstar))

        _, _, bstar = lax.while_loop(
            thr_cond, thr_body,
            (jnp.int32(NBVEC - 1), jnp.int32(0), jnp.int32(0)))

        @plsc.parallel_loop(0, CVEC, unroll=4)
        def _init(j):
            ck_v[pl.ds(j * 16, 16)] = zeros16 - 1
            ci_v[pl.ds(j * 16, 16)] = zeros16

        @plsc.parallel_loop(0, NVEC, unroll=4, carry=jnp.int32(0))
        def _compact(i, off):
            k = plsc.bitcast(row_v[i >> 3, pl.ds((i & 7) * 16, 16)],
                             jnp.int32)
            m = (k >> SHIFT) >= bstar
            offc = jnp.minimum(off, CAND - 16)
            plsc.store_compressed(ck_v.at[pl.ds(offc, 16)], k, mask=m)
            plsc.store_compressed(ci_v.at[pl.ds(offc, 16)], iota + i * 16,
                                  mask=m)
            return off + jnp.max(plsc.all_reduce_population_count(m))

        for st in range(1, 10):
            kk = 1 << st
            j = kk >> 1
            while j >= 1:
                if j >= 16:
                    jv = j // 16

                    @plsc.parallel_loop(0, CVEC // 2, unroll=2)
                    def _cross(t, jv=jv, kk=kk):
                        q = t // jv
                        v = q * (2 * jv) + (t - q * jv)
                        p = v + jv
                        ka = ck_v[pl.ds(v * 16, 16)]
                        ia = ci_v[pl.ds(v * 16, 16)]
                        kb = ck_v[pl.ds(p * 16, 16)]
                        ib = ci_v[pl.ds(p * 16, 16)]
                        a_first = (ka > kb) | ((ka == kb) & (ia < ib))
                        dirf = ((zeros16 + v * 16) & kk) == 0
                        keep = jnp.where(dirf, a_first, ~a_first)
                        ck_v[pl.ds(v * 16, 16)] = jnp.where(keep, ka, kb)
                        ci_v[pl.ds(v * 16, 16)] = jnp.where(keep, ia, ib)
                        ck_v[pl.ds(p * 16, 16)] = jnp.where(keep, kb, ka)
                        ci_v[pl.ds(p * 16, 16)] = jnp.where(keep, ib, ia)
                else:
                    perm = iota ^ j
                    lower = (iota & j) == 0

                    @plsc.parallel_loop(0, CVEC, unroll=2)
                    def _intra(v, j=j, kk=kk, perm=perm, lower=lower):
                        ks = ck_v[pl.ds(v * 16, 16)]
                        is_ = ci_v[pl.ds(v * 16, 16)]
                        ko = plsc.load_gather(ck_v, [v * 16 + perm])
                        io = plsc.load_gather(ci_v, [v * 16 + perm])
                        s_first = (ks > ko) | ((ks == ko) & (is_ < io))
                        dirf = ((iota + v * 16) & kk) == 0
                        keep = jnp.where(lower == dirf, s_first, ~s_first)
                        ck_v[pl.ds(v * 16, 16)] = jnp.where(keep, ks, ko)
                        ci_v[pl.ds(v * 16, 16)] = jnp.where(keep, is_, io)
                j >>= 1

        s0 = scale_v[0, pl.ds(0, 16)]
        s1 = scale_v[0, pl.ds(16, 16)]
        s2 = scale_v[0, pl.ds(32, 16)]
        s3 = scale_v[0, pl.ds(48, 16)]

        @plsc.parallel_loop(0, OUT_PAD // 16, unroll=2)
        def _emit(jj):
            r2 = jj >> 3
            cc = (jj & 7) * 16
            kj = ck_v[pl.ds(jj * 16, 16)]
            ij = ci_v[pl.ds(jj * 16, 16)]
            sco_v[r2, pl.ds(cc, 16)] = plsc.bitcast(kj, jnp.float32)
            bq = ij // NCLS
            lab_v[r2, pl.ds(cc, 16)] = ij - bq * NCLS
            bqc = jnp.minimum(bq, QC // NCLS - 1)
            b4 = bqc * 4
            br = b4 >> 7
            bc = b4 & 127
            cx = plsc.load_gather(boxes_v, [br, bc])
            cy = plsc.load_gather(boxes_v, [br, bc + 1])
            w = plsc.load_gather(boxes_v, [br, bc + 2])
            h = plsc.load_gather(boxes_v, [br, bc + 3])
            bxo_v[r2, pl.ds(cc, 16)] = (cx - 0.5 * w) * s0
            bxo_v[3 + r2, pl.ds(cc, 16)] = (cy - 0.5 * h) * s1
            bxo_v[6 + r2, pl.ds(cc, 16)] = (cx + 0.5 * w) * s2
            bxo_v[9 + r2, pl.ds(cc, 16)] = (cy + 0.5 * h) * s3

        pltpu.sync_copy(sco_v, scores_hbm.at[row])
        pltpu.sync_copy(lab_v, labels_hbm.at[row])
        pltpu.sync_copy(bxo_v, boxeso_hbm.at[row])
        return 0

    lax.fori_loop(0, 2, do_row, 0)


def kernel(obj_logits, obj_boxes, target_sizes):
    B, Q, C = obj_logits.shape
    flat = obj_logits.reshape(B, Q * C)
    prob = pl.pallas_call(
        _sig_body,
        out_shape=jax.ShapeDtypeStruct((B, QC_PAD // 128, 128), jnp.float32),
        grid=(B // 8,),
        in_specs=[pl.BlockSpec((8, QC), lambda i: (i, 0))],
        out_specs=pl.BlockSpec((8, QC_PAD // 128, 128), lambda i: (i, 0, 0)),
    )(flat)

    boxes = jnp.pad(obj_boxes.reshape(B, 4 * Q),
                    ((0, 0), (0, BOX_PAD - 4 * Q)))
    boxes = boxes.reshape(B, BOX_PAD // 128, 128)

    img_h = target_sizes[:, 0].astype(jnp.float32)
    img_w = target_sizes[:, 1].astype(jnp.float32)
    scale = jnp.stack([img_w, img_h, img_w, img_h], axis=1)
    scale = jnp.repeat(scale, 16, axis=1)
    scale = jnp.pad(scale, ((0, 0), (0, 64))).reshape(B, 1, 128)

    sc = pl.kernel(
        _sc_topk,
        out_type=[
            jax.ShapeDtypeStruct((NROW, 3, 128), jnp.float32),
            jax.ShapeDtypeStruct((NROW, 3, 128), jnp.int32),
            jax.ShapeDtypeStruct((NROW, 12, 128), jnp.float32),
        ],
        mesh=plsc.VectorSubcoreMesh(core_axis_name="c", subcore_axis_name="s"),
        compiler_params=pltpu.CompilerParams(
            needs_layout_passes=False, use_tc_tiling_on_sc=True),
        scratch_types=[
            pltpu.VMEM((QC_PAD // 128, 128), jnp.float32),
            pltpu.VMEM((NBUCKET,), jnp.int32),
            pltpu.VMEM((CAND,), jnp.int32),
            pltpu.VMEM((CAND,), jnp.int32),
            pltpu.VMEM((BOX_PAD // 128, 128), jnp.float32),
            pltpu.VMEM((1, 128), jnp.float32),
            pltpu.VMEM((3, 128), jnp.float32),
            pltpu.VMEM((3, 128), jnp.int32),
            pltpu.VMEM((12, 128), jnp.float32),
        ],
    )
    scores_p, labels_p, boxes_p = sc(prob, boxes, scale)
    scores = scores_p.reshape(NROW, OUT_PAD)[:, :NSEL]
    labels = labels_p.reshape(NROW, OUT_PAD)[:, :NSEL]
    boxes_o = boxes_p.reshape(NROW, 4, OUT_PAD).transpose(0, 2, 1)[:, :NSEL, :]
    return scores, labels, boxes_o


# speedup vs baseline: 1.9162x; 1.7080x over previous
import jax
import jax.numpy as jnp
from jax import lax
from jax.experimental import pallas as pl
from jax.experimental.pallas import tpu as pltpu
from jax.experimental.pallas import tpu_sc as plsc

NSEL = 300
NCLS = 91
QC = 81900
QC_PAD = 81920
NVEC = QC_PAD // 16
NVEC2 = NCLS * 64
SHIFT = 16
NBUCKET = 16384
NBVEC = NBUCKET // 16
CAND = 512
CVEC = CAND // 16
OUT_PAD = 384
NROW = 64
BOX_PAD = 3712


def _sig_body(x_ref, o_ref):
    s = jax.nn.sigmoid(x_ref[...])
    o_ref[:, :, :7, :] = s[:, :, :896].reshape(NCLS, 8, 7, 128)
    o_ref[:, :, 7, :4] = s[:, :, 896:]
    o_ref[:, :, 7, 4:] = jnp.zeros((NCLS, 8, 124), jnp.float32)


def _sc_topk(prob_hbm, boxes_hbm, scale_hbm, scores_hbm, labels_hbm,
             boxeso_hbm, row_v, hist_v, ck_v, ci_v, boxes_v, scale_v,
             sco_v, lab_v, bxo_v):
    wid = lax.axis_index("s") * 2 + lax.axis_index("c")
    iota = lax.iota(jnp.int32, 16)
    zeros16 = iota * 0
    ones16 = zeros16 + 1

    def do_row(rr, _):
        row = wid * 2 + rr
        pltpu.sync_copy(prob_hbm.at[:, row], row_v)
        pltpu.sync_copy(boxes_hbm.at[row], boxes_v)
        pltpu.sync_copy(scale_hbm.at[row], scale_v)

        @plsc.parallel_loop(0, NBVEC, unroll=8)
        def _zero(j):
            hist_v[pl.ds(j * 16, 16)] = zeros16

        @plsc.parallel_loop(0, NVEC2, unroll=8)
        def _hist(i):
            j = i & 63
            k = plsc.bitcast(row_v[i >> 6, j >> 3, pl.ds((j & 7) * 16, 16)],
                             jnp.int32)
            plsc.addupdate_scatter(hist_v, [k >> SHIFT], ones16)

        def thr_cond(carry):
            jr, acc, _ = carry
            return (acc < NSEL) & (jr >= 0)

        def thr_body(carry):
            jr, acc, bstar = carry
            h = hist_v[pl.ds(jr * 16, 16)]
            s = jnp.sum(h)
            rc = plsc.cumsum(lax.rev(h, (0,)))
            f = jnp.max(plsc.all_reduce_ffs((acc + rc) >= NSEL))
            cand_b = jr * 16 + 15 - f
            cross = (acc + s) >= NSEL
            return (jr - 1, acc + s, jnp.where(cross, cand_b, bstar))

        _, _, bstar = lax.while_loop(
            thr_cond, thr_body,
            (jnp.int32(NBVEC - 1), jnp.int32(0), jnp.int32(0)))

        @plsc.parallel_loop(0, CVEC, unroll=4)
        def _init(j):
            ck_v[pl.ds(j * 16, 16)] = zeros16 - 1
            ci_v[pl.ds(j * 16, 16)] = zeros16

        @plsc.parallel_loop(0, NVEC2, unroll=4, carry=jnp.int32(0))
        def _compact(i, off):
            j = i & 63
            k = plsc.bitcast(row_v[i >> 6, j >> 3, pl.ds((j & 7) * 16, 16)],
                             jnp.int32)
            m = (k >> SHIFT) >= bstar
            offc = jnp.minimum(off, CAND - 16)
            idx = (j * 16 + iota) * NCLS + (i >> 6)
            plsc.store_compressed(ck_v.at[pl.ds(offc, 16)], k, mask=m)
            plsc.store_compressed(ci_v.at[pl.ds(offc, 16)], idx, mask=m)
            return off + jnp.max(plsc.all_reduce_population_count(m))

        for st in range(1, 10):
            kk = 1 << st
            j = kk >> 1
            while j >= 1:
                if j >= 16:
                    jv = j // 16

                    @plsc.parallel_loop(0, CVEC // 2, unroll=2)
                    def _cross(t, jv=jv, kk=kk):
                        q = t // jv
                        v = q * (2 * jv) + (t - q * jv)
                        p = v + jv
                        ka = ck_v[pl.ds(v * 16, 16)]
                        ia = ci_v[pl.ds(v * 16, 16)]
                        kb = ck_v[pl.ds(p * 16, 16)]
                        ib = ci_v[pl.ds(p * 16, 16)]
                        a_first = (ka > kb) | ((ka == kb) & (ia < ib))
                        dirf = ((zeros16 + v * 16) & kk) == 0
                        keep = jnp.where(dirf, a_first, ~a_first)
                        ck_v[pl.ds(v * 16, 16)] = jnp.where(keep, ka, kb)
                        ci_v[pl.ds(v * 16, 16)] = jnp.where(keep, ia, ib)
                        ck_v[pl.ds(p * 16, 16)] = jnp.where(keep, kb, ka)
                        ci_v[pl.ds(p * 16, 16)] = jnp.where(keep, ib, ia)
                else:
                    perm = iota ^ j
                    lower = (iota & j) == 0

                    @plsc.parallel_loop(0, CVEC, unroll=2)
                    def _intra(v, j=j, kk=kk, perm=perm, lower=lower):
                        ks = ck_v[pl.ds(v * 16, 16)]
                        is_ = ci_v[pl.ds(v * 16, 16)]
                        ko = plsc.load_gather(ck_v, [v * 16 + perm])
                        io = plsc.load_gather(ci_v, [v * 16 + perm])
                        s_first = (ks > ko) | ((ks == ko) & (is_ < io))
                        dirf = ((iota + v * 16) & kk) == 0
                        keep = jnp.where(lower == dirf, s_first, ~s_first)
                        ck_v[pl.ds(v * 16, 16)] = jnp.where(keep, ks, ko)
                        ci_v[pl.ds(v * 16, 16)] = jnp.where(keep, is_, io)
                j >>= 1

        s0 = scale_v[0, pl.ds(0, 16)]
        s1 = scale_v[0, pl.ds(16, 16)]
        s2 = scale_v[0, pl.ds(32, 16)]
        s3 = scale_v[0, pl.ds(48, 16)]

        @plsc.parallel_loop(0, OUT_PAD // 16, unroll=2)
        def _emit(jj):
            r2 = jj >> 3
            cc = (jj & 7) * 16
            kj = ck_v[pl.ds(jj * 16, 16)]
            ij = ci_v[pl.ds(jj * 16, 16)]
            sco_v[r2, pl.ds(cc, 16)] = plsc.bitcast(kj, jnp.float32)
            bq = ij // NCLS
            lab_v[r2, pl.ds(cc, 16)] = ij - bq * NCLS
            bqc = jnp.minimum(bq, QC // NCLS - 1)
            b4 = bqc * 4
            br = b4 >> 7
            bc = b4 & 127
            cx = plsc.load_gather(boxes_v, [br, bc])
            cy = plsc.load_gather(boxes_v, [br, bc + 1])
            w = plsc.load_gather(boxes_v, [br, bc + 2])
            h = plsc.load_gather(boxes_v, [br, bc + 3])
            bxo_v[r2, pl.ds(cc, 16)] = (cx - 0.5 * w) * s0
            bxo_v[3 + r2, pl.ds(cc, 16)] = (cy - 0.5 * h) * s1
            bxo_v[6 + r2, pl.ds(cc, 16)] = (cx + 0.5 * w) * s2
            bxo_v[9 + r2, pl.ds(cc, 16)] = (cy + 0.5 * h) * s3

        pltpu.sync_copy(sco_v, scores_hbm.at[row])
        pltpu.sync_copy(lab_v, labels_hbm.at[row])
        pltpu.sync_copy(bxo_v, boxeso_hbm.at[row])
        return 0

    lax.fori_loop(0, 2, do_row, 0)


def kernel(obj_logits, obj_boxes, target_sizes):
    B, Q, C = obj_logits.shape
    lg = obj_logits.transpose(2, 0, 1)
    prob = pl.pallas_call(
        _sig_body,
        out_shape=jax.ShapeDtypeStruct((C, B, 8, 128), jnp.float32),
        grid=(B // 8,),
        in_specs=[pl.BlockSpec((C, 8, Q), lambda i: (0, i, 0))],
        out_specs=pl.BlockSpec((C, 8, 8, 128), lambda i: (0, i, 0, 0)),
    )(lg)

    boxes = jnp.pad(obj_boxes.reshape(B, 4 * Q),
                    ((0, 0), (0, BOX_PAD - 4 * Q)))
    boxes = boxes.reshape(B, BOX_PAD // 128, 128)

    img_h = target_sizes[:, 0].astype(jnp.float32)
    img_w = target_sizes[:, 1].astype(jnp.float32)
    scale = jnp.stack([img_w, img_h, img_w, img_h], axis=1)
    scale = jnp.repeat(scale, 16, axis=1)
    scale = jnp.pad(scale, ((0, 0), (0, 64))).reshape(B, 1, 128)

    sc = pl.kernel(
        _sc_topk,
        out_type=[
            jax.ShapeDtypeStruct((NROW, 3, 128), jnp.float32),
            jax.ShapeDtypeStruct((NROW, 3, 128), jnp.int32),
            jax.ShapeDtypeStruct((NROW, 12, 128), jnp.float32),
        ],
        mesh=plsc.VectorSubcoreMesh(core_axis_name="c", subcore_axis_name="s"),
        compiler_params=pltpu.CompilerParams(
            needs_layout_passes=False, use_tc_tiling_on_sc=True),
        scratch_types=[
            pltpu.VMEM((NCLS, 8, 128), jnp.float32),
            pltpu.VMEM((NBUCKET,), jnp.int32),
            pltpu.VMEM((CAND,), jnp.int32),
            pltpu.VMEM((CAND,), jnp.int32),
            pltpu.VMEM((BOX_PAD // 128, 128), jnp.float32),
            pltpu.VMEM((1, 128), jnp.float32),
            pltpu.VMEM((3, 128), jnp.float32),
            pltpu.VMEM((3, 128), jnp.int32),
            pltpu.VMEM((12, 128), jnp.float32),
        ],
    )
    scores_p, labels_p, boxes_p = sc(prob, boxes, scale)
    scores = scores_p.reshape(NROW, OUT_PAD)[:, :NSEL]
    labels = labels_p.reshape(NROW, OUT_PAD)[:, :NSEL]
    boxes_o = boxes_p.reshape(NROW, 4, OUT_PAD).transpose(0, 2, 1)[:, :NSEL, :]
    return scores, labels, boxes_o


# speedup vs baseline: 2.0119x; 1.0499x over previous
import jax
import jax.numpy as jnp
from jax import lax
from jax.experimental import pallas as pl
from jax.experimental.pallas import tpu as pltpu
from jax.experimental.pallas import tpu_sc as plsc

NSEL = 300
NCLS = 91
QC = 81900
QC_PAD = 81920
NVEC = QC_PAD // 16
NVEC2 = NCLS * 64
SHIFT = 16
NBUCKET = 16384
NBVEC = NBUCKET // 16
CAND = 512
CVEC = CAND // 16
OUT_PAD = 384
NROW = 64
BOX_PAD = 3712


def _prep_body(lg_ref, bx_ref, ts_ref, o_ref, bo_ref, sc_ref):
    s = jax.nn.sigmoid(lg_ref[...])
    o_ref[:, :, :7, :] = s[:, :, :896].reshape(NCLS, 8, 7, 128)
    o_ref[:, :, 7, :4] = s[:, :, 896:]
    o_ref[:, :, 7, 4:] = jnp.zeros((NCLS, 8, 124), jnp.float32)
    b = bx_ref[...]
    bo_ref[:, :, :7, :] = b[:, :, :896].reshape(8, 4, 7, 128)
    bo_ref[:, :, 7, :4] = b[:, :, 896:]
    bo_ref[:, :, 7, 4:] = jnp.zeros((8, 4, 124), jnp.float32)
    w16 = jnp.broadcast_to(ts_ref[:, 1:2].astype(jnp.float32), (8, 16))
    h16 = jnp.broadcast_to(ts_ref[:, 0:1].astype(jnp.float32), (8, 16))
    sc_ref[:, 0, :] = jnp.concatenate(
        [w16, h16, w16, h16, jnp.zeros((8, 64), jnp.float32)], axis=1)


def _sc_topk(prob_hbm, boxes_hbm, scale_hbm, scores_hbm, labels_hbm,
             boxeso_hbm, row_v, hist_v, ck_v, ci_v, boxes_v, scale_v,
             sco_v, lab_v, bxo_v):
    wid = lax.axis_index("s") * 2 + lax.axis_index("c")
    iota = lax.iota(jnp.int32, 16)
    zeros16 = iota * 0
    ones16 = zeros16 + 1

    def do_row(rr, _):
        row = wid * 2 + rr
        pltpu.sync_copy(prob_hbm.at[:, row], row_v)
        pltpu.sync_copy(boxes_hbm.at[row], boxes_v)
        pltpu.sync_copy(scale_hbm.at[row], scale_v)

        @plsc.parallel_loop(0, NBVEC, unroll=8)
        def _zero(j):
            hist_v[pl.ds(j * 16, 16)] = zeros16

        @plsc.parallel_loop(0, NVEC2, unroll=8)
        def _hist(i):
            j = i & 63
            k = plsc.bitcast(row_v[i >> 6, j >> 3, pl.ds((j & 7) * 16, 16)],
                             jnp.int32)
            plsc.addupdate_scatter(hist_v, [k >> SHIFT], ones16)

        def thr_cond(carry):
            jr, acc, _ = carry
            return (acc < NSEL) & (jr >= 0)

        def thr_body(carry):
            jr, acc, bstar = carry
            h = hist_v[pl.ds(jr * 16, 16)]
            s = jnp.sum(h)
            rc = plsc.cumsum(lax.rev(h, (0,)))
            f = jnp.max(plsc.all_reduce_ffs((acc + rc) >= NSEL))
            cand_b = jr * 16 + 15 - f
            cross = (acc + s) >= NSEL
            return (jr - 1, acc + s, jnp.where(cross, cand_b, bstar))

        _, _, bstar = lax.while_loop(
            thr_cond, thr_body,
            (jnp.int32(NBVEC - 1), jnp.int32(0), jnp.int32(0)))

        @plsc.parallel_loop(0, CVEC, unroll=4)
        def _init(j):
            ck_v[pl.ds(j * 16, 16)] = zeros16 - 1
            ci_v[pl.ds(j * 16, 16)] = zeros16

        @plsc.parallel_loop(0, NVEC2, unroll=8, carry=jnp.int32(0))
        def _compact(i, off):
            j = i & 63
            k = plsc.bitcast(row_v[i >> 6, j >> 3, pl.ds((j & 7) * 16, 16)],
                             jnp.int32)
            m = (k >> SHIFT) >= bstar
            offc = jnp.minimum(off, CAND - 16)
            idx = (j * 16 + iota) * NCLS + (i >> 6)
            plsc.store_compressed(ck_v.at[pl.ds(offc, 16)], k, mask=m)
            plsc.store_compressed(ci_v.at[pl.ds(offc, 16)], idx, mask=m)
            return off + jnp.max(plsc.all_reduce_population_count(m))

        for st in range(1, 10):
            kk = 1 << st
            j = kk >> 1
            while j >= 1:
                if j >= 16:
                    jv = j // 16

                    @plsc.parallel_loop(0, CVEC // 2, unroll=2)
                    def _cross(t, jv=jv, kk=kk):
                        q = t // jv
                        v = q * (2 * jv) + (t - q * jv)
                        p = v + jv
                        ka = ck_v[pl.ds(v * 16, 16)]
                        ia = ci_v[pl.ds(v * 16, 16)]
                        kb = ck_v[pl.ds(p * 16, 16)]
                        ib = ci_v[pl.ds(p * 16, 16)]
                        a_first = (ka > kb) | ((ka == kb) & (ia < ib))
                        dirf = ((zeros16 + v * 16) & kk) == 0
                        keep = jnp.where(dirf, a_first, ~a_first)
                        ck_v[pl.ds(v * 16, 16)] = jnp.where(keep, ka, kb)
                        ci_v[pl.ds(v * 16, 16)] = jnp.where(keep, ia, ib)
                        ck_v[pl.ds(p * 16, 16)] = jnp.where(keep, kb, ka)
                        ci_v[pl.ds(p * 16, 16)] = jnp.where(keep, ib, ia)
                else:
                    perm = iota ^ j
                    lower = (iota & j) == 0

                    @plsc.parallel_loop(0, CVEC, unroll=2)
                    def _intra(v, j=j, kk=kk, perm=perm, lower=lower):
                        ks = ck_v[pl.ds(v * 16, 16)]
                        is_ = ci_v[pl.ds(v * 16, 16)]
                        ko = plsc.load_gather(ck_v, [v * 16 + perm])
                        io = plsc.load_gather(ci_v, [v * 16 + perm])
                        s_first = (ks > ko) | ((ks == ko) & (is_ < io))
                        dirf = ((iota + v * 16) & kk) == 0
                        keep = jnp.where(lower == dirf, s_first, ~s_first)
                        ck_v[pl.ds(v * 16, 16)] = jnp.where(keep, ks, ko)
                        ci_v[pl.ds(v * 16, 16)] = jnp.where(keep, is_, io)
                j >>= 1

        s0 = scale_v[0, pl.ds(0, 16)]
        s1 = scale_v[0, pl.ds(16, 16)]
        s2 = scale_v[0, pl.ds(32, 16)]
        s3 = scale_v[0, pl.ds(48, 16)]

        @plsc.parallel_loop(0, OUT_PAD // 16, unroll=2)
        def _emit(jj):
            r2 = jj >> 3
            cc = (jj & 7) * 16
            kj = ck_v[pl.ds(jj * 16, 16)]
            ij = ci_v[pl.ds(jj * 16, 16)]
            sco_v[r2, pl.ds(cc, 16)] = plsc.bitcast(kj, jnp.float32)
            bq = ij // NCLS
            lab_v[r2, pl.ds(cc, 16)] = ij - bq * NCLS
            bqc = jnp.minimum(bq, QC // NCLS - 1)
            br = bqc >> 7
            bc = bqc & 127
            cx = plsc.load_gather(boxes_v, [zeros16, br, bc])
            cy = plsc.load_gather(boxes_v, [ones16, br, bc])
            w = plsc.load_gather(boxes_v, [zeros16 + 2, br, bc])
            h = plsc.load_gather(boxes_v, [zeros16 + 3, br, bc])
            bxo_v[r2, pl.ds(cc, 16)] = (cx - 0.5 * w) * s0
            bxo_v[3 + r2, pl.ds(cc, 16)] = (cy - 0.5 * h) * s1
            bxo_v[6 + r2, pl.ds(cc, 16)] = (cx + 0.5 * w) * s2
            bxo_v[9 + r2, pl.ds(cc, 16)] = (cy + 0.5 * h) * s3

        pltpu.sync_copy(sco_v, scores_hbm.at[row])
        pltpu.sync_copy(lab_v, labels_hbm.at[row])
        pltpu.sync_copy(bxo_v, boxeso_hbm.at[row])
        return 0

    lax.fori_loop(0, 2, do_row, 0)


def kernel(obj_logits, obj_boxes, target_sizes):
    B, Q, C = obj_logits.shape
    lg = obj_logits.transpose(2, 0, 1)
    bxt = obj_boxes.transpose(0, 2, 1)
    prob, boxes, scale = pl.pallas_call(
        _prep_body,
        out_shape=[
            jax.ShapeDtypeStruct((C, B, 8, 128), jnp.float32),
            jax.ShapeDtypeStruct((B, 4, 8, 128), jnp.float32),
            jax.ShapeDtypeStruct((B, 1, 128), jnp.float32),
        ],
        grid=(B // 8,),
        in_specs=[
            pl.BlockSpec((C, 8, Q), lambda i: (0, i, 0)),
            pl.BlockSpec((8, 4, Q), lambda i: (i, 0, 0)),
            pl.BlockSpec((8, 2), lambda i: (i, 0)),
        ],
        out_specs=[
            pl.BlockSpec((C, 8, 8, 128), lambda i: (0, i, 0, 0)),
            pl.BlockSpec((8, 4, 8, 128), lambda i: (i, 0, 0, 0)),
            pl.BlockSpec((8, 1, 128), lambda i: (i, 0, 0)),
        ],
    )(lg, bxt, target_sizes)

    sc = pl.kernel(
        _sc_topk,
        out_type=[
            jax.ShapeDtypeStruct((NROW, 3, 128), jnp.float32),
            jax.ShapeDtypeStruct((NROW, 3, 128), jnp.int32),
            jax.ShapeDtypeStruct((NROW, 12, 128), jnp.float32),
        ],
        mesh=plsc.VectorSubcoreMesh(core_axis_name="c", subcore_axis_name="s"),
        compiler_params=pltpu.CompilerParams(
            needs_layout_passes=False, use_tc_tiling_on_sc=True),
        scratch_types=[
            pltpu.VMEM((NCLS, 8, 128), jnp.float32),
            pltpu.VMEM((NBUCKET,), jnp.int32),
            pltpu.VMEM((CAND,), jnp.int32),
            pltpu.VMEM((CAND,), jnp.int32),
            pltpu.VMEM((4, 8, 128), jnp.float32),
            pltpu.VMEM((1, 128), jnp.float32),
            pltpu.VMEM((3, 128), jnp.float32),
            pltpu.VMEM((3, 128), jnp.int32),
            pltpu.VMEM((12, 128), jnp.float32),
        ],
    )
    scores_p, labels_p, boxes_p = sc(prob, boxes, scale)
    scores = scores_p.reshape(NROW, OUT_PAD)[:, :NSEL]
    labels = labels_p.reshape(NROW, OUT_PAD)[:, :NSEL]
    boxes_o = boxes_p.reshape(NROW, 4, OUT_PAD).transpose(0, 2, 1)[:, :NSEL, :]
    return scores, labels, boxes_o


# speedup vs baseline: 2.0465x; 1.0172x over previous
import jax
import jax.numpy as jnp
from jax import lax
from jax.experimental import pallas as pl
from jax.experimental.pallas import tpu as pltpu
from jax.experimental.pallas import tpu_sc as plsc

NSEL = 300
NCLS = 91
QC = 81900
QC_PAD = 81920
NVEC = QC_PAD // 16
NVEC2 = NCLS * 64
SHIFT = 16
NBUCKET = 16384
NBVEC = NBUCKET // 16
CAND = 512
CVEC = CAND // 16
OUT_PAD = 384
NROW = 64
BOX_PAD = 3712


def _prep_body(lg_ref, bx_ref, ts_ref, o_ref, bo_ref, sc_ref):
    s = jax.nn.sigmoid(lg_ref[...])
    o_ref[:, :, :7, :] = s[:, :, :896].reshape(NCLS, 8, 7, 128)
    o_ref[:, :, 7, :4] = s[:, :, 896:]
    o_ref[:, :, 7, 4:] = jnp.zeros((NCLS, 8, 124), jnp.float32)
    b = bx_ref[...]
    bo_ref[:, :, :7, :] = b[:, :, :896].reshape(8, 4, 7, 128)
    bo_ref[:, :, 7, :4] = b[:, :, 896:]
    bo_ref[:, :, 7, 4:] = jnp.zeros((8, 4, 124), jnp.float32)
    w16 = jnp.broadcast_to(ts_ref[:, 1:2].astype(jnp.float32), (8, 16))
    h16 = jnp.broadcast_to(ts_ref[:, 0:1].astype(jnp.float32), (8, 16))
    sc_ref[:, 0, :] = jnp.concatenate(
        [w16, h16, w16, h16, jnp.zeros((8, 64), jnp.float32)], axis=1)


CHUNKS = ((0, 23), (23, 23), (46, 23), (69, 22))


def _sc_topk(prob_hbm, boxes_hbm, scale_hbm, scores_hbm, labels_hbm,
             boxeso_hbm, row_v, hist_v, ck_v, ci_v, boxes_v, scale_v,
             sco_v, lab_v, bxo_v, sem0, sem1, sem2, sem3):
    wid = lax.axis_index("s") * 2 + lax.axis_index("c")
    iota = lax.iota(jnp.int32, 16)
    zeros16 = iota * 0
    ones16 = zeros16 + 1
    sems = (sem0, sem1, sem2, sem3)

    def issue(row):
        return [
            pltpu.async_copy(prob_hbm.at[pl.ds(st, sz), row],
                             row_v.at[pl.ds(st, sz)], sems[ci])
            for ci, (st, sz) in enumerate(CHUNKS)
        ]

    descs = issue(wid * 2)
    for rr in range(2):
        row = wid * 2 + rr
        pltpu.sync_copy(boxes_hbm.at[row], boxes_v)
        pltpu.sync_copy(scale_hbm.at[row], scale_v)

        @plsc.parallel_loop(0, NBVEC, unroll=8)
        def _zero(j):
            hist_v[pl.ds(j * 16, 16)] = zeros16

        for ci, (st, sz) in enumerate(CHUNKS):
            descs[ci].wait()

            @plsc.parallel_loop(st * 64, (st + sz) * 64, unroll=8)
            def _hist(i):
                j = i & 63
                k = plsc.bitcast(
                    row_v[i >> 6, j >> 3, pl.ds((j & 7) * 16, 16)],
                    jnp.int32)
                plsc.addupdate_scatter(hist_v, [k >> SHIFT], ones16)

        def thr_cond(carry):
            jr, acc, _ = carry
            return (acc < NSEL) & (jr >= 0)

        def thr_body(carry):
            jr, acc, bstar = carry
            h = hist_v[pl.ds(jr * 16, 16)]
            s = jnp.sum(h)
            rc = plsc.cumsum(lax.rev(h, (0,)))
            f = jnp.max(plsc.all_reduce_ffs((acc + rc) >= NSEL))
            cand_b = jr * 16 + 15 - f
            cross = (acc + s) >= NSEL
            return (jr - 1, acc + s, jnp.where(cross, cand_b, bstar))

        _, _, bstar = lax.while_loop(
            thr_cond, thr_body,
            (jnp.int32(NBVEC - 1), jnp.int32(0), jnp.int32(0)))

        @plsc.parallel_loop(0, CVEC, unroll=4)
        def _init(j):
            ck_v[pl.ds(j * 16, 16)] = zeros16 - 1
            ci_v[pl.ds(j * 16, 16)] = zeros16

        @plsc.parallel_loop(0, NVEC2, unroll=8, carry=jnp.int32(0))
        def _compact(i, off):
            j = i & 63
            k = plsc.bitcast(row_v[i >> 6, j >> 3, pl.ds((j & 7) * 16, 16)],
                             jnp.int32)
            m = (k >> SHIFT) >= bstar
            offc = jnp.minimum(off, CAND - 16)
            idx = (j * 16 + iota) * NCLS + (i >> 6)
            plsc.store_compressed(ck_v.at[pl.ds(offc, 16)], k, mask=m)
            plsc.store_compressed(ci_v.at[pl.ds(offc, 16)], idx, mask=m)
            return off + jnp.max(plsc.all_reduce_population_count(m))

        if rr == 0:
            descs = issue(wid * 2 + 1)

        for st in range(1, 10):
            kk = 1 << st
            j = kk >> 1
            while j >= 1:
                if j >= 16:
                    jv = j // 16

                    @plsc.parallel_loop(0, CVEC // 2, unroll=2)
                    def _cross(t, jv=jv, kk=kk):
                        q = t // jv
                        v = q * (2 * jv) + (t - q * jv)
                        p = v + jv
                        ka = ck_v[pl.ds(v * 16, 16)]
                        ia = ci_v[pl.ds(v * 16, 16)]
                        kb = ck_v[pl.ds(p * 16, 16)]
                        ib = ci_v[pl.ds(p * 16, 16)]
                        a_first = (ka > kb) | ((ka == kb) & (ia < ib))
                        dirf = ((zeros16 + v * 16) & kk) == 0
                        keep = jnp.where(dirf, a_first, ~a_first)
                        ck_v[pl.ds(v * 16, 16)] = jnp.where(keep, ka, kb)
                        ci_v[pl.ds(v * 16, 16)] = jnp.where(keep, ia, ib)
                        ck_v[pl.ds(p * 16, 16)] = jnp.where(keep, kb, ka)
                        ci_v[pl.ds(p * 16, 16)] = jnp.where(keep, ib, ia)
                else:
                    perm = iota ^ j
                    lower = (iota & j) == 0

                    @plsc.parallel_loop(0, CVEC, unroll=2)
                    def _intra(v, j=j, kk=kk, perm=perm, lower=lower):
                        ks = ck_v[pl.ds(v * 16, 16)]
                        is_ = ci_v[pl.ds(v * 16, 16)]
                        ko = plsc.load_gather(ck_v, [v * 16 + perm])
                        io = plsc.load_gather(ci_v, [v * 16 + perm])
                        s_first = (ks > ko) | ((ks == ko) & (is_ < io))
                        dirf = ((iota + v * 16) & kk) == 0
                        keep = jnp.where(lower == dirf, s_first, ~s_first)
                        ck_v[pl.ds(v * 16, 16)] = jnp.where(keep, ks, ko)
                        ci_v[pl.ds(v * 16, 16)] = jnp.where(keep, is_, io)
                j >>= 1

        s0 = scale_v[0, pl.ds(0, 16)]
        s1 = scale_v[0, pl.ds(16, 16)]
        s2 = scale_v[0, pl.ds(32, 16)]
        s3 = scale_v[0, pl.ds(48, 16)]

        @plsc.parallel_loop(0, OUT_PAD // 16, unroll=2)
        def _emit(jj):
            r2 = jj >> 3
            cc = (jj & 7) * 16
            kj = ck_v[pl.ds(jj * 16, 16)]
            ij = ci_v[pl.ds(jj * 16, 16)]
            sco_v[r2, pl.ds(cc, 16)] = plsc.bitcast(kj, jnp.float32)
            bq = ij // NCLS
            lab_v[r2, pl.ds(cc, 16)] = ij - bq * NCLS
            bqc = jnp.minimum(bq, QC // NCLS - 1)
            br = bqc >> 7
            bc = bqc & 127
            cx = plsc.load_gather(boxes_v, [zeros16, br, bc])
            cy = plsc.load_gather(boxes_v, [ones16, br, bc])
            w = plsc.load_gather(boxes_v, [zeros16 + 2, br, bc])
            h = plsc.load_gather(boxes_v, [zeros16 + 3, br, bc])
            bxo_v[r2, pl.ds(cc, 16)] = (cx - 0.5 * w) * s0
            bxo_v[3 + r2, pl.ds(cc, 16)] = (cy - 0.5 * h) * s1
            bxo_v[6 + r2, pl.ds(cc, 16)] = (cx + 0.5 * w) * s2
            bxo_v[9 + r2, pl.ds(cc, 16)] = (cy + 0.5 * h) * s3

        pltpu.sync_copy(sco_v, scores_hbm.at[row])
        pltpu.sync_copy(lab_v, labels_hbm.at[row])
        pltpu.sync_copy(bxo_v, boxeso_hbm.at[row])


def kernel(obj_logits, obj_boxes, target_sizes):
    B, Q, C = obj_logits.shape
    lg = obj_logits.transpose(2, 0, 1)
    bxt = obj_boxes.transpose(0, 2, 1)
    prob, boxes, scale = pl.pallas_call(
        _prep_body,
        out_shape=[
            jax.ShapeDtypeStruct((C, B, 8, 128), jnp.float32),
            jax.ShapeDtypeStruct((B, 4, 8, 128), jnp.float32),
            jax.ShapeDtypeStruct((B, 1, 128), jnp.float32),
        ],
        grid=(B // 8,),
        in_specs=[
            pl.BlockSpec((C, 8, Q), lambda i: (0, i, 0)),
            pl.BlockSpec((8, 4, Q), lambda i: (i, 0, 0)),
            pl.BlockSpec((8, 2), lambda i: (i, 0)),
        ],
        out_specs=[
            pl.BlockSpec((C, 8, 8, 128), lambda i: (0, i, 0, 0)),
            pl.BlockSpec((8, 4, 8, 128), lambda i: (i, 0, 0, 0)),
            pl.BlockSpec((8, 1, 128), lambda i: (i, 0, 0)),
        ],
    )(lg, bxt, target_sizes)

    sc = pl.kernel(
        _sc_topk,
        out_type=[
            jax.ShapeDtypeStruct((NROW, 3, 128), jnp.float32),
            jax.ShapeDtypeStruct((NROW, 3, 128), jnp.int32),
            jax.ShapeDtypeStruct((NROW, 12, 128), jnp.float32),
        ],
        mesh=plsc.VectorSubcoreMesh(core_axis_name="c", subcore_axis_name="s"),
        compiler_params=pltpu.CompilerParams(
            needs_layout_passes=False, use_tc_tiling_on_sc=True),
        scratch_types=[
            pltpu.VMEM((NCLS, 8, 128), jnp.float32),
            pltpu.VMEM((NBUCKET,), jnp.int32),
            pltpu.VMEM((CAND,), jnp.int32),
            pltpu.VMEM((CAND,), jnp.int32),
            pltpu.VMEM((4, 8, 128), jnp.float32),
            pltpu.VMEM((1, 128), jnp.float32),
            pltpu.VMEM((3, 128), jnp.float32),
            pltpu.VMEM((3, 128), jnp.int32),
            pltpu.VMEM((12, 128), jnp.float32),
            pltpu.SemaphoreType.DMA,
            pltpu.SemaphoreType.DMA,
            pltpu.SemaphoreType.DMA,
            pltpu.SemaphoreType.DMA,
        ],
    )
    scores_p, labels_p, boxes_p = sc(prob, boxes, scale)
    scores = scores_p.reshape(NROW, OUT_PAD)[:, :NSEL]
    labels = labels_p.reshape(NROW, OUT_PAD)[:, :NSEL]
    boxes_o = boxes_p.reshape(NROW, 4, OUT_PAD).transpose(0, 2, 1)[:, :NSEL, :]
    return scores, labels, boxes_o
